# baseline (speedup 1.0000x reference)
import functools
import math

import jax
import jax.numpy as jnp
import numpy as np
from jax import lax
from jax.experimental import pallas as pl
from jax.experimental.pallas import tpu as pltpu
from jax.experimental.pallas import tpu_sc as plsc

D_MODEL = 512
SEQ = 200
B_ROWS = 1024

NUM_WORKERS = 32
TOK_PER_W = B_ROWS * SEQ // NUM_WORKERS
LANES = 16
CPR = D_MODEL // LANES

GROUP = 32
NGROUPS = TOK_PER_W // GROUP
NWIN = 25
GPW = NGROUPS // NWIN
NBUF = 5
DP = 2


def _pe_table() -> np.ndarray:
    pe = np.zeros((SEQ, D_MODEL), dtype=np.float32)
    position = np.arange(0, SEQ, dtype=np.float32)[:, None]
    div_term = np.exp(
        np.arange(0, D_MODEL, 2, dtype=np.float32) * -(math.log(10000.0) / D_MODEL)
    )
    pe[:, 0::2] = np.sin(position * div_term)
    pe[:, 1::2] = np.cos(position * div_term)
    return pe


_PE2 = np.concatenate([_pe_table(), _pe_table()], axis=0)

_MESH = plsc.VectorSubcoreMesh(core_axis_name="c", subcore_axis_name="s")


@functools.partial(
    pl.kernel,
    out_type=jax.ShapeDtypeStruct((B_ROWS * SEQ, D_MODEL), jnp.float32),
    mesh=_MESH,
    scratch_types=[
        pltpu.VMEM((TOK_PER_W,), jnp.int32),
        pltpu.VMEM((GROUP, D_MODEL), jnp.float32),
        pltpu.VMEM((NBUF, GROUP, D_MODEL), jnp.float32),
    ] + [pltpu.SemaphoreType.DMA] * (2 * NBUF),
)
def _emb_kernel(idx_hbm, table_hbm, pe2_hbm, out_hbm, idx_v, pe_v, G, *sems):
    gs = sems[:NBUF]
    ss = sems[NBUF:]
    wid = lax.axis_index("s") * 2 + lax.axis_index("c")
    tok0 = wid * TOK_PER_W
    pltpu.sync_copy(idx_hbm.at[pl.ds(tok0, TOK_PER_W)], idx_v)

    def g_of(i):
        return (i // GPW) + NWIN * (i % GPW)

    def issue_gather(i, slot):
        g = g_of(i)
        pltpu.async_copy(
            table_hbm.at[idx_v.at[pl.ds(g * GROUP, GROUP)]], G.at[slot], gs[slot]
        )

    def wait_gather(slot):
        pltpu.make_async_copy(
            table_hbm.at[pl.ds(0, GROUP), :], G.at[slot], gs[slot]
        ).wait()

    def drain_store(slot):
        pltpu.make_async_copy(
            G.at[slot], out_hbm.at[pl.ds(0, GROUP), :], ss[slot]
        ).wait()

    for i0 in range(DP):
        issue_gather(i0, i0)

    def outer(go, carry):
        for b in range(NBUF):
            i = go * NBUF + b
            j = i // GPW
            m = i - j * GPW

            @pl.when(m == 0)
            def _reload_pe():
                w0 = lax.rem(j * GROUP, SEQ)
                pltpu.sync_copy(pe2_hbm.at[pl.ds(w0, GROUP), :], pe_v)

            wait_gather(b)

            def add_row(rr, c):
                for u in range(CPR):
                    s = pl.ds(u * LANES, LANES)
                    G[b, rr, s] = G[b, rr, s] + pe_v[rr, s]
                return c

            lax.fori_loop(0, GROUP, add_row, 0)

            g = g_of(i)
            pltpu.async_copy(
                G.at[b], out_hbm.at[pl.ds(tok0 + g * GROUP, GROUP), :], ss[b]
            )

            ip = i + DP
            sp = (b + DP) % NBUF

            @pl.when(ip < NGROUPS)
            def _prefetch():
                @pl.when(ip >= NBUF)
                def _drain():
                    drain_store(sp)

                issue_gather(ip, sp)

        return carry

    lax.fori_loop(0, NGROUPS // NBUF, outer, 0)

    for b in range(NBUF):
        drain_store(b)


def kernel(x, table):
    idx = x.reshape(-1).astype(jnp.int32)
    pe2 = jnp.asarray(_PE2)
    out = _emb_kernel(idx, table.astype(jnp.float32), pe2)
    return out.reshape(x.shape[0], x.shape[1], D_MODEL)
